# baseline (speedup 1.0000x reference)
import jax
import jax.numpy as jnp
from jax import lax
from jax.experimental import pallas as pl
from jax.experimental.pallas import tpu as pltpu
from jax.experimental.pallas import tpu_sc as plsc

B, L, D = 4096, 200, 32
NBINS = 10
NC, NS, LANES = 2, 16, 16
NW = NC * NS
ROWS_PER_W = B // NW
BLK = ROWS_PER_W * L
LPAD = 208
NGROUPS = LPAD // LANES
ACC = NBINS * D
CHUNKS = ((0, 128), (128, 80))


def _sc_embedder(t_hbm, idx_hbm, wx_hbm, ww_hbm, out_hbm,
                 t_all, idx_all, ww_v, emb_v, acc_v, cnt_v, sem_g, sem_o):
    wid = lax.axis_index("s") * NC + lax.axis_index("c")
    base = wid * BLK

    def gather_descs(i, buf):
        ds = []
        for off, sz in CHUNKS:
            idxsl = idx_all.at[pl.ds(i * L + off, sz)]
            ds.append(pltpu.make_async_copy(
                wx_hbm.at[idxsl], emb_v[buf].at[pl.ds(off, sz)], sem_g[buf]))
            ds.append(pltpu.make_async_copy(
                ww_hbm.at[idxsl], ww_v[buf].at[pl.ds(off, sz)], sem_g[buf]))
        return ds

    def fire(i, buf):
        for d in gather_descs(i, buf):
            d.start()

    def wait(i, buf):
        for d in gather_descs(i, buf):
            d.wait()

    def out_desc(i, buf):
        return pltpu.make_async_copy(
            acc_v[buf].at[pl.ds(0, ACC)],
            out_hbm.at[pl.ds((wid * ROWS_PER_W + i) * ACC, ACC)], sem_o[buf])

    def compute(i, buf):
        acc = acc_v[buf]
        cnt = cnt_v[buf]

        @plsc.parallel_loop(0, 2 * ACC // LANES, unroll=8)
        def _zero_body(r):
            acc[pl.ds(r * LANES, LANES)] = jnp.zeros((LANES,), jnp.float32)
        cnt[pl.ds(0, LANES)] = jnp.zeros((LANES,), jnp.float32)
        cnt[pl.ds(LANES, LANES)] = jnp.zeros((LANES,), jnp.float32)
        wait(i, buf)
        ones = jnp.ones((LANES,), jnp.float32)
        tb = i * L
        for g in range(NGROUPS):
            bank = (g % 2) * ACC
            cbank = (g % 2) * LANES
            toks = lax.iota(jnp.int32, LANES) + g * LANES
            t16 = t_all[pl.ds(tb + g * LANES, LANES)]
            bins = t16.astype(jnp.int32)
            mask = t16 < 10.0
            if (g + 1) * LANES > L:
                mask = mask & (toks < L)
            w16 = jnp.exp(ww_v[buf][pl.ds(g * LANES, LANES)])
            plsc.addupdate_scatter(cnt, [bins + (cbank + 1)], ones, mask=mask)
            bins32 = bins * D + bank
            @plsc.parallel_loop(0, D, unroll=16)
            def _dim_body(d):
                dfull = jnp.full((LANES,), d, jnp.int32)
                vals = plsc.load_gather(emb_v[buf], [toks, dfull])
                plsc.addupdate_scatter(acc, [bins32 + dfull], vals * w16,
                                       mask=mask)

        @plsc.parallel_loop(0, NBINS)
        def _div_body(bn):
            c0 = plsc.load_gather(cnt, [jnp.full((LANES,), 1, jnp.int32) + bn])
            c1 = plsc.load_gather(
                cnt, [jnp.full((LANES,), LANES + 1, jnp.int32) + bn])
            denom = c0 + c1 + 1e-6
            for h in range(2):
                off = bn * D + h * LANES
                acc[pl.ds(off, LANES)] = (
                    acc[pl.ds(off, LANES)] + acc[pl.ds(ACC + off, LANES)]
                ) / denom
        out_desc(i, buf).start()

    idx_all[pl.ds(BLK - 8, LANES)] = jnp.zeros((LANES,), jnp.int32)
    pltpu.sync_copy(t_hbm.at[pl.ds(base, BLK)], t_all.at[pl.ds(0, BLK)])
    pltpu.sync_copy(idx_hbm.at[pl.ds(base, BLK)], idx_all.at[pl.ds(0, BLK)])

    fire(0, 0)

    def pair_body(p, carry):
        i0 = 2 * p
        i1 = i0 + 1
        fire(i1, 1)

        @pl.when(p > 0)
        def _():
            out_desc(i0 - 2, 0).wait()
        compute(i0, 0)

        @pl.when(p < ROWS_PER_W // 2 - 1)
        def _():
            fire(i0 + 2, 0)

        @pl.when(p > 0)
        def _():
            out_desc(i1 - 2, 1).wait()
        compute(i1, 1)
        return carry

    lax.fori_loop(0, ROWS_PER_W // 2, pair_body, 0)
    out_desc(ROWS_PER_W - 2, 0).wait()
    out_desc(ROWS_PER_W - 1, 1).wait()


def _build():
    mesh = plsc.VectorSubcoreMesh(core_axis_name="c", subcore_axis_name="s")
    return pl.kernel(
        _sc_embedder,
        out_type=jax.ShapeDtypeStruct((B * NBINS * D,), jnp.float32),
        mesh=mesh,
        scratch_types=[
            pltpu.VMEM((BLK + 8,), jnp.float32),
            pltpu.VMEM((BLK + 8,), jnp.int32),
            [pltpu.VMEM((LPAD,), jnp.float32)] * 2,
            [pltpu.VMEM((LPAD, D), jnp.float32)] * 2,
            [pltpu.VMEM((2 * ACC,), jnp.float32)] * 2,
            [pltpu.VMEM((2 * LANES,), jnp.float32)] * 2,
            [pltpu.SemaphoreType.DMA] * 2,
            [pltpu.SemaphoreType.DMA] * 2,
        ],
        compiler_params=pltpu.CompilerParams(needs_layout_passes=False,
                                             use_tc_tiling_on_sc=False),
    )


def kernel(X, Wx, Ww):
    T = X[:, :, 0].reshape(-1)
    idx = X[:, :, 1].astype(jnp.int32).reshape(-1)
    fn = _build()
    out = fn(T, idx, Wx, Ww.reshape(-1))
    return out.reshape(B, NBINS, D)


# speedup vs baseline: 1.9532x; 1.9532x over previous
import jax
import jax.numpy as jnp
from jax import lax
from jax.experimental import pallas as pl
from jax.experimental.pallas import tpu as pltpu
from jax.experimental.pallas import tpu_sc as plsc

B, L, D = 4096, 200, 32
NBINS = 10
NSLOT = NBINS + 1
NC, NS, LANES = 2, 16, 16
NW = NC * NS
ROWS_PER_W = B // NW
BLK = ROWS_PER_W * L
NGROUPS = 13
WOFF = LANES
GCHUNKS = ((0, 128), (128, 80))


def _sc_embedder(t_hbm, idx_hbm, wx_hbm, ww_hbm, out_hbm,
                 t_all, idx_all, ww_v, emb_v, sbin_v, res_v, cnt_v,
                 zero_v, shared, sem_g, sem_s, sem_o):
    wid = lax.axis_index("s") * NC + lax.axis_index("c")
    wtile = lax.axis_index("s")
    base = wid * BLK
    sbase = wtile * NSLOT

    def gather_descs(i, buf):
        ds = []
        for off, sz in GCHUNKS:
            idxsl = idx_all.at[pl.ds(i * L + off, sz)]
            ds.append(pltpu.make_async_copy(
                wx_hbm.at[idxsl], emb_v[buf].at[pl.ds(off, sz)], sem_g[buf]))
            ds.append(pltpu.make_async_copy(
                ww_hbm.at[idxsl], ww_v[buf].at[pl.ds(WOFF + off, sz)],
                sem_g[buf]))
        return ds

    def scatter_descs(buf):
        return [
            pltpu.make_async_copy(
                emb_v[buf].at[pl.ds(0, 128)],
                shared.at[sbin_v[buf].at[0]], sem_s),
            pltpu.make_async_copy(
                emb_v[buf].at[pl.ds(128, 128)],
                shared.at[sbin_v[buf].at[1]], sem_s),
        ]

    def out_desc(i, buf):
        return pltpu.make_async_copy(
            res_v[buf].at[pl.ds(0, NBINS)],
            out_hbm.at[pl.ds((wid * ROWS_PER_W + i) * NBINS, NBINS)],
            sem_o[buf])

    def compute(i, buf):
        cnt = cnt_v[buf]
        cnt[...] = jnp.zeros((LANES,), jnp.float32)
        wait_gathers = gather_descs(i, buf)
        for dsc in wait_gathers:
            dsc.wait()
        ones = jnp.ones((LANES,), jnp.float32)
        tb = i * L
        woff16 = jnp.zeros((LANES,), jnp.int32) + wtile * NSLOT
        for g in range(NGROUPS):
            toks = lax.iota(jnp.int32, LANES) + g * LANES
            t16 = t_all[pl.ds(tb + g * LANES, LANES)]
            bins = t16.astype(jnp.int32)
            mask = t16 < 10.0
            if (g + 1) * LANES > L:
                mask = mask & (toks < L)
                bins = jnp.where(toks < L, bins, NBINS)
            plsc.addupdate_scatter(cnt, [bins + 1], ones, mask=mask)
            r, c = divmod(g * LANES, 128)
            sbin_v[buf][r, pl.ds(c, LANES)] = bins + woff16
            wsl = ww_v[buf].at[pl.ds(WOFF + g * LANES, LANES)]
            wsl[...] = jnp.exp(wsl[...])
        for c in (80, 96, 112):
            sbin_v[buf][1, pl.ds(c, LANES)] = woff16 + NBINS

        @plsc.parallel_loop(0, L, unroll=8)
        def _scale_body(tok):
            w = plsc.load_gather(
                ww_v[buf], [jnp.zeros((LANES,), jnp.int32) + (tok + WOFF)])
            emb_v[buf][tok, pl.ds(0, LANES)] = (
                emb_v[buf][tok, pl.ds(0, LANES)] * w)
            emb_v[buf][tok, pl.ds(LANES, LANES)] = (
                emb_v[buf][tok, pl.ds(LANES, LANES)] * w)

        for dsc in scatter_descs(buf):
            dsc.start(add=True)
            dsc.wait()
        pltpu.sync_copy(shared.at[pl.ds(sbase, NBINS)],
                        res_v[buf].at[pl.ds(0, NBINS)])
        pltpu.sync_copy(zero_v, shared.at[pl.ds(sbase, NSLOT)])

        @plsc.parallel_loop(0, NBINS)
        def _div_body(bn):
            c0 = plsc.load_gather(cnt, [jnp.zeros((LANES,), jnp.int32) + bn + 1])
            denom = c0 + 1e-6
            for h in range(2):
                res_v[buf][bn, pl.ds(h * LANES, LANES)] = (
                    res_v[buf][bn, pl.ds(h * LANES, LANES)] / denom)
        out_desc(i, buf).start()

    idx_all[pl.ds(BLK - 8, LANES)] = jnp.zeros((LANES,), jnp.int32)
    pltpu.sync_copy(t_hbm.at[pl.ds(base, BLK)], t_all.at[pl.ds(0, BLK)])
    pltpu.sync_copy(idx_hbm.at[pl.ds(base, BLK)], idx_all.at[pl.ds(0, BLK)])
    zf = jnp.zeros((LANES,), jnp.float32)
    for r in range(NSLOT):
        zero_v[r, pl.ds(0, LANES)] = zf
        zero_v[r, pl.ds(LANES, LANES)] = zf
    pltpu.sync_copy(zero_v, shared.at[pl.ds(sbase, NSLOT)])

    for dsc in gather_descs(0, 0):
        dsc.start()

    def pair_body(p, carry):
        i0 = 2 * p
        i1 = i0 + 1
        for dsc in gather_descs(i1, 1):
            dsc.start()

        @pl.when(p > 0)
        def _():
            out_desc(i0 - 2, 0).wait()
        compute(i0, 0)

        @pl.when(p < ROWS_PER_W // 2 - 1)
        def _():
            for dsc in gather_descs(i0 + 2, 0):
                dsc.start()

        @pl.when(p > 0)
        def _():
            out_desc(i1 - 2, 1).wait()
        compute(i1, 1)
        return carry

    lax.fori_loop(0, ROWS_PER_W // 2, pair_body, 0)
    out_desc(ROWS_PER_W - 2, 0).wait()
    out_desc(ROWS_PER_W - 1, 1).wait()


def _build():
    mesh = plsc.VectorSubcoreMesh(core_axis_name="c", subcore_axis_name="s")
    return pl.kernel(
        _sc_embedder,
        out_type=jax.ShapeDtypeStruct((B * NBINS, D), jnp.float32),
        mesh=mesh,
        scratch_types=[
            pltpu.VMEM((BLK + 8,), jnp.float32),
            pltpu.VMEM((BLK + 8,), jnp.int32),
            [pltpu.VMEM((256 + WOFF,), jnp.float32)] * 2,
            [pltpu.VMEM((256, D), jnp.float32)] * 2,
            [pltpu.VMEM((2, 128), jnp.int32)] * 2,
            [pltpu.VMEM((NBINS, D), jnp.float32)] * 2,
            [pltpu.VMEM((LANES,), jnp.float32)] * 2,
            pltpu.VMEM((NSLOT, D), jnp.float32),
            pltpu.VMEM_SHARED((NS * NSLOT, D), jnp.float32),
            [pltpu.SemaphoreType.DMA] * 2,
            pltpu.SemaphoreType.DMA,
            [pltpu.SemaphoreType.DMA] * 2,
        ],
        compiler_params=pltpu.CompilerParams(needs_layout_passes=False,
                                             use_tc_tiling_on_sc=False),
    )


def kernel(X, Wx, Ww):
    T = X[:, :, 0].reshape(-1)
    idx = X[:, :, 1].astype(jnp.int32).reshape(-1)
    fn = _build()
    out = fn(T, idx, Wx, Ww.reshape(-1))
    return out.reshape(B, NBINS, D)
